# Initial kernel scaffold; baseline (speedup 1.0000x reference)
import jax
import jax.numpy as jnp
from jax.experimental import pallas as pl


def kernel(ctrl_tokens, embed_table, W, b):
    raise NotImplementedError("write your pallas kernel here")


# speedup vs baseline: 2.8961x; 2.8961x over previous
import functools

import jax
import jax.numpy as jnp
from jax import lax
from jax.experimental import pallas as pl
from jax.experimental.pallas import tpu as pltpu
from jax.experimental.pallas import tpu_sc as plsc

D = 768
E = 192
V = 64
NTOK = 4 * 8192
NW = 32
TOK_PER_W = NTOK // NW
G = 32
NCHUNK = TOK_PER_W // G
A_BLK = 8


def _tables_body(et_ref, w_ref, b_ref, ft01_ref, ft23_ref, s_ref):
    i = pl.program_id(0)

    @pl.when(i == 0)
    def _():
        for s in range(4):
            s_ref[s] = lax.dot_general(
                et_ref[:], w_ref[:, s * E:(s + 1) * E],
                (((1,), (1,)), ((), ())),
                preferred_element_type=jnp.float32)

    a0 = s_ref[0, pl.ds(i * A_BLK, A_BLK)]
    a2 = s_ref[2, pl.ds(i * A_BLK, A_BLK)]
    c1 = s_ref[1] + b_ref[:]
    c3 = s_ref[3]
    ft01_ref[:] = a0[:, None, :] + c1[None, :, :]
    ft23_ref[:] = a2[:, None, :] + c3[None, :, :]


def _make_tables(et, w, b2):
    grid = V // A_BLK
    return pl.pallas_call(
        _tables_body,
        grid=(grid,),
        in_specs=[
            pl.BlockSpec((V, E), lambda i: (0, 0)),
            pl.BlockSpec((D, D), lambda i: (0, 0)),
            pl.BlockSpec((1, D), lambda i: (0, 0)),
        ],
        out_specs=[
            pl.BlockSpec((A_BLK, V, D), lambda i: (i, 0, 0)),
            pl.BlockSpec((A_BLK, V, D), lambda i: (i, 0, 0)),
        ],
        out_shape=[
            jax.ShapeDtypeStruct((V, V, D), jnp.float32),
            jax.ShapeDtypeStruct((V, V, D), jnp.float32),
        ],
        scratch_shapes=[pltpu.VMEM((4, V, D), jnp.float32)],
    )(et, w, b2)


def _sc_body(t0_hbm, t1_hbm, t2_hbm, t3_hbm, ft01_hbm, ft23_hbm, out_hbm,
             t0_v, t1_v, t2_v, t3_v, i01_v, i23_v,
             st1_v, st2_v):
    cid = lax.axis_index("c")
    sid = lax.axis_index("s")
    wid = sid * 2 + cid
    iota = lax.iota(jnp.int32, 16)


    def chunk(g, _):
        tok_base = wid * TOK_PER_W + g * G
        pltpu.sync_copy(t0_hbm.at[pl.ds(tok_base, G)], t0_v)
        pltpu.sync_copy(t1_hbm.at[pl.ds(tok_base, G)], t1_v)
        pltpu.sync_copy(t2_hbm.at[pl.ds(tok_base, G)], t2_v)
        pltpu.sync_copy(t3_hbm.at[pl.ds(tok_base, G)], t3_v)
        for i in range(G // 16):
            ds = pl.ds(16 * i, 16)
            i01_v[ds] = t0_v[ds] * 64 + t1_v[ds]
            i23_v[ds] = t2_v[ds] * 64 + t3_v[ds]
        pltpu.sync_copy(ft01_hbm.at[i01_v], st1_v)
        pltpu.sync_copy(ft23_hbm.at[i23_v], st2_v)

        def addrow(t, _):
            for j in range(D // 16):
                ds = pl.ds(16 * j, 16)
                plsc.addupdate(st1_v.at[t, ds], st2_v[t, ds])
            return 0

        lax.fori_loop(0, G, addrow, 0)
        pltpu.sync_copy(st1_v, out_hbm.at[pl.ds(tok_base, G)])
        return 0

    lax.fori_loop(0, NCHUNK, chunk, 0)


@functools.lru_cache(maxsize=1)
def _sc_lookup():
    return pl.kernel(
        _sc_body,
        out_type=jax.ShapeDtypeStruct((NTOK, D), jnp.float32),
        mesh=plsc.VectorSubcoreMesh(core_axis_name="c", subcore_axis_name="s"),
        scratch_types=[
            pltpu.VMEM((G,), jnp.int32),
            pltpu.VMEM((G,), jnp.int32),
            pltpu.VMEM((G,), jnp.int32),
            pltpu.VMEM((G,), jnp.int32),
            pltpu.VMEM((G,), jnp.int32),
            pltpu.VMEM((G,), jnp.int32),
            pltpu.VMEM((G, D), jnp.float32),
            pltpu.VMEM((G, D), jnp.float32),
        ],
    )


def kernel(ctrl_tokens, embed_table, W, b):
    Bc, Tc, _ = ctrl_tokens.shape
    ft01, ft23 = _make_tables(embed_table, W, b.reshape(1, D))
    tf = ctrl_tokens.reshape(-1, 4)
    out = _sc_lookup()(tf[:, 0], tf[:, 1], tf[:, 2], tf[:, 3],
                       ft01.reshape(V * V, D), ft23.reshape(V * V, D))
    return out.reshape(Bc, Tc, D)


# speedup vs baseline: 5.6810x; 1.9616x over previous
import functools

import jax
import jax.numpy as jnp
from jax import lax
from jax.experimental import pallas as pl
from jax.experimental.pallas import tpu as pltpu
from jax.experimental.pallas import tpu_sc as plsc

D = 768
E = 192
V = 64
NTOK = 4 * 8192
NW = 32
TOK_PER_W = NTOK // NW
G = 32
NCHUNK = TOK_PER_W // G
A_BLK = 8


def _tables_body(et_ref, w_ref, b_ref, ft01_ref, ft23_ref, s_ref):
    i = pl.program_id(0)

    @pl.when(i == 0)
    def _():
        for s in range(4):
            s_ref[s] = lax.dot_general(
                et_ref[:], w_ref[:, s * E:(s + 1) * E],
                (((1,), (1,)), ((), ())),
                preferred_element_type=jnp.float32)

    a0 = s_ref[0, pl.ds(i * A_BLK, A_BLK)]
    a2 = s_ref[2, pl.ds(i * A_BLK, A_BLK)]
    c1 = s_ref[1] + b_ref[:]
    c3 = s_ref[3]
    ft01_ref[:] = a0[:, None, :] + c1[None, :, :]
    ft23_ref[:] = a2[:, None, :] + c3[None, :, :]


def _make_tables(et, w, b2):
    grid = V // A_BLK
    return pl.pallas_call(
        _tables_body,
        grid=(grid,),
        in_specs=[
            pl.BlockSpec((V, E), lambda i: (0, 0)),
            pl.BlockSpec((D, D), lambda i: (0, 0)),
            pl.BlockSpec((1, D), lambda i: (0, 0)),
        ],
        out_specs=[
            pl.BlockSpec((A_BLK, V, D), lambda i: (i, 0, 0)),
            pl.BlockSpec((A_BLK, V, D), lambda i: (i, 0, 0)),
        ],
        out_shape=[
            jax.ShapeDtypeStruct((V, V, D), jnp.float32),
            jax.ShapeDtypeStruct((V, V, D), jnp.float32),
        ],
        scratch_shapes=[pltpu.VMEM((4, V, D), jnp.float32)],
    )(et, w, b2)


def _sc_body(t0_hbm, t1_hbm, t2_hbm, t3_hbm, ft01_hbm, ft23_hbm, out_hbm,
             t0_v, t1_v, t2_v, t3_v, i01_v, i23_v,
             st1a_v, st2a_v, st1b_v, st2b_v, gsem0, gsem1, osem0, osem1):
    cid = lax.axis_index("c")
    sid = lax.axis_index("s")
    wid = sid * 2 + cid
    base = wid * TOK_PER_W

    pltpu.sync_copy(t0_hbm.at[pl.ds(base, TOK_PER_W)], t0_v)
    pltpu.sync_copy(t1_hbm.at[pl.ds(base, TOK_PER_W)], t1_v)
    pltpu.sync_copy(t2_hbm.at[pl.ds(base, TOK_PER_W)], t2_v)
    pltpu.sync_copy(t3_hbm.at[pl.ds(base, TOK_PER_W)], t3_v)

    def ibody(i, _):
        ds = pl.ds(i * 16, 16)
        i01_v[ds] = t0_v[ds] * 64 + t1_v[ds]
        i23_v[ds] = t2_v[ds] * 64 + t3_v[ds]
        return 0

    lax.fori_loop(0, TOK_PER_W // 16, ibody, 0)

    st1 = (st1a_v, st1b_v)
    st2 = (st2a_v, st2b_v)
    gsem = (gsem0, gsem1)
    osem = (osem0, osem1)

    def fire_gathers(c, b):
        cc = jnp.where(c < NCHUNK, c, 0)
        idx01 = i01_v.at[pl.ds(cc * G, G)]
        idx23 = i23_v.at[pl.ds(cc * G, G)]
        pltpu.async_copy(ft01_hbm.at[idx01], st1[b], gsem[b])
        pltpu.async_copy(ft23_hbm.at[idx23], st2[b], gsem[b])

    def drain_gathers(b):
        pltpu.make_async_copy(ft01_hbm.at[pl.ds(0, G)], st1[b], gsem[b]).wait()
        pltpu.make_async_copy(ft23_hbm.at[pl.ds(0, G)], st2[b], gsem[b]).wait()

    def drain_store(b):
        pltpu.make_async_copy(
            st1[b], out_hbm.at[pl.ds(base, G)], osem[b]).wait()

    def addrows(b):
        def addrow(t, _):
            for j in range(D // 16):
                ds = pl.ds(16 * j, 16)
                plsc.addupdate(st1[b].at[t, ds], st2[b][t, ds])
            return 0

        lax.fori_loop(0, G, addrow, 0)

    fire_gathers(0, 0)
    pltpu.async_copy(st1b_v, out_hbm.at[pl.ds(base + G, G)], osem1)

    def pair(i, _):
        c0 = 2 * i
        drain_gathers(0)
        drain_store(1)
        fire_gathers(c0 + 1, 1)
        addrows(0)
        s0 = pltpu.async_copy(
            st1a_v, out_hbm.at[pl.ds(base + c0 * G, G)], osem0)
        drain_gathers(1)
        s0.wait()
        fire_gathers(c0 + 2, 0)
        addrows(1)
        pltpu.async_copy(
            st1b_v, out_hbm.at[pl.ds(base + (c0 + 1) * G, G)], osem1)
        return 0

    lax.fori_loop(0, NCHUNK // 2, pair, 0)
    drain_gathers(0)
    drain_store(1)


@functools.lru_cache(maxsize=1)
def _sc_lookup():
    return pl.kernel(
        _sc_body,
        out_type=jax.ShapeDtypeStruct((NTOK, D), jnp.float32),
        mesh=plsc.VectorSubcoreMesh(core_axis_name="c", subcore_axis_name="s"),
        scratch_types=[
            pltpu.VMEM((TOK_PER_W,), jnp.int32),
            pltpu.VMEM((TOK_PER_W,), jnp.int32),
            pltpu.VMEM((TOK_PER_W,), jnp.int32),
            pltpu.VMEM((TOK_PER_W,), jnp.int32),
            pltpu.VMEM((TOK_PER_W,), jnp.int32),
            pltpu.VMEM((TOK_PER_W,), jnp.int32),
            pltpu.VMEM((G, D), jnp.float32),
            pltpu.VMEM((G, D), jnp.float32),
            pltpu.VMEM((G, D), jnp.float32),
            pltpu.VMEM((G, D), jnp.float32),
            pltpu.SemaphoreType.DMA,
            pltpu.SemaphoreType.DMA,
            pltpu.SemaphoreType.DMA,
            pltpu.SemaphoreType.DMA,
        ],
    )


def kernel(ctrl_tokens, embed_table, W, b):
    Bc, Tc, _ = ctrl_tokens.shape
    ft01, ft23 = _make_tables(embed_table, W, b.reshape(1, D))
    tf = ctrl_tokens.reshape(-1, 4)
    out = _sc_lookup()(tf[:, 0], tf[:, 1], tf[:, 2], tf[:, 3],
                       ft01.reshape(V * V, D), ft23.reshape(V * V, D))
    return out.reshape(Bc, Tc, D)


# speedup vs baseline: 5.8144x; 1.0235x over previous
import functools

import jax
import jax.numpy as jnp
from jax import lax
from jax.experimental import pallas as pl
from jax.experimental.pallas import tpu as pltpu
from jax.experimental.pallas import tpu_sc as plsc

D = 768
E = 192
V = 64
NTOK = 4 * 8192
NW = 32
TOK_PER_W = NTOK // NW
G = 16
NCHUNK = TOK_PER_W // G
NBUF = 4
A_BLK = 8


def _tables_body(et_ref, w_ref, b_ref, ft01_ref, ft23_ref, s_ref):
    i = pl.program_id(0)

    @pl.when(i == 0)
    def _():
        for s in range(4):
            s_ref[s] = lax.dot_general(
                et_ref[:], w_ref[:, s * E:(s + 1) * E],
                (((1,), (1,)), ((), ())),
                preferred_element_type=jnp.float32)

    a0 = s_ref[0, pl.ds(i * A_BLK, A_BLK)]
    a2 = s_ref[2, pl.ds(i * A_BLK, A_BLK)]
    c1 = s_ref[1] + b_ref[:]
    c3 = s_ref[3]
    ft01_ref[:] = a0[:, None, :] + c1[None, :, :]
    ft23_ref[:] = a2[:, None, :] + c3[None, :, :]


def _make_tables(et, w, b2):
    grid = V // A_BLK
    return pl.pallas_call(
        _tables_body,
        grid=(grid,),
        in_specs=[
            pl.BlockSpec((V, E), lambda i: (0, 0)),
            pl.BlockSpec((D, D), lambda i: (0, 0)),
            pl.BlockSpec((1, D), lambda i: (0, 0)),
        ],
        out_specs=[
            pl.BlockSpec((A_BLK, V, D), lambda i: (i, 0, 0)),
            pl.BlockSpec((A_BLK, V, D), lambda i: (i, 0, 0)),
        ],
        out_shape=[
            jax.ShapeDtypeStruct((V, V, D), jnp.float32),
            jax.ShapeDtypeStruct((V, V, D), jnp.float32),
        ],
        scratch_shapes=[pltpu.VMEM((4, V, D), jnp.float32)],
    )(et, w, b2)


def _sc_body(t0_hbm, t1_hbm, t2_hbm, t3_hbm, ft01_hbm, ft23_hbm, out_hbm,
             t0_v, t1_v, t2_v, t3_v, i01_v, i23_v,
             st1a_v, st2a_v, st1b_v, st2b_v,
             st1c_v, st2c_v, st1d_v, st2d_v,
             gsem0, gsem1, gsem2, gsem3, osem0, osem1, osem2, osem3):
    cid = lax.axis_index("c")
    sid = lax.axis_index("s")
    wid = sid * 2 + cid
    base = wid * TOK_PER_W

    pltpu.sync_copy(t0_hbm.at[pl.ds(base, TOK_PER_W)], t0_v)
    pltpu.sync_copy(t1_hbm.at[pl.ds(base, TOK_PER_W)], t1_v)
    pltpu.sync_copy(t2_hbm.at[pl.ds(base, TOK_PER_W)], t2_v)
    pltpu.sync_copy(t3_hbm.at[pl.ds(base, TOK_PER_W)], t3_v)

    def ibody(i, _):
        ds = pl.ds(i * 16, 16)
        i01_v[ds] = t0_v[ds] * 64 + t1_v[ds]
        i23_v[ds] = t2_v[ds] * 64 + t3_v[ds]
        return 0

    lax.fori_loop(0, TOK_PER_W // 16, ibody, 0)

    st1 = (st1a_v, st1b_v, st1c_v, st1d_v)
    st2 = (st2a_v, st2b_v, st2c_v, st2d_v)
    gsem = (gsem0, gsem1, gsem2, gsem3)
    osem = (osem0, osem1, osem2, osem3)

    def fire_gathers(c, b):
        cc = jnp.where(c < NCHUNK, c, 0)
        idx01 = i01_v.at[pl.ds(cc * G, G)]
        idx23 = i23_v.at[pl.ds(cc * G, G)]
        pltpu.async_copy(ft01_hbm.at[idx01], st1[b], gsem[b])
        pltpu.async_copy(ft23_hbm.at[idx23], st2[b], gsem[b])

    def drain_gathers(b):
        pltpu.make_async_copy(ft01_hbm.at[pl.ds(0, G)], st1[b], gsem[b]).wait()
        pltpu.make_async_copy(ft23_hbm.at[pl.ds(0, G)], st2[b], gsem[b]).wait()

    def drain_store(b):
        pltpu.make_async_copy(
            st1[b], out_hbm.at[pl.ds(base, G)], osem[b]).wait()

    def addrows(b):
        def addrow(t, _):
            for j in range(D // 16):
                ds = pl.ds(16 * j, 16)
                plsc.addupdate(st1[b].at[t, ds], st2[b][t, ds])
            return 0

        lax.fori_loop(0, G, addrow, 0)

    fire_gathers(0, 0)
    fire_gathers(1, 1)
    pltpu.async_copy(st1c_v, out_hbm.at[pl.ds(base + 2 * G, G)], osem2)
    pltpu.async_copy(st1d_v, out_hbm.at[pl.ds(base + 3 * G, G)], osem3)

    def quad(i, _):
        c0 = 4 * i
        for b in range(NBUF):
            c = c0 + b
            b2 = (b + 2) % NBUF
            drain_gathers(b)
            drain_store(b2)
            fire_gathers(c + 2, b2)
            addrows(b)
            pltpu.async_copy(
                st1[b], out_hbm.at[pl.ds(base + c * G, G)], osem[b])
        return 0

    lax.fori_loop(0, NCHUNK // NBUF, quad, 0)
    drain_gathers(0)
    drain_gathers(1)
    drain_store(2)
    drain_store(3)


@functools.lru_cache(maxsize=1)
def _sc_lookup():
    return pl.kernel(
        _sc_body,
        out_type=jax.ShapeDtypeStruct((NTOK, D), jnp.float32),
        mesh=plsc.VectorSubcoreMesh(core_axis_name="c", subcore_axis_name="s"),
        scratch_types=[
            pltpu.VMEM((TOK_PER_W,), jnp.int32),
            pltpu.VMEM((TOK_PER_W,), jnp.int32),
            pltpu.VMEM((TOK_PER_W,), jnp.int32),
            pltpu.VMEM((TOK_PER_W,), jnp.int32),
            pltpu.VMEM((TOK_PER_W,), jnp.int32),
            pltpu.VMEM((TOK_PER_W,), jnp.int32),
            pltpu.VMEM((G, D), jnp.float32),
            pltpu.VMEM((G, D), jnp.float32),
            pltpu.VMEM((G, D), jnp.float32),
            pltpu.VMEM((G, D), jnp.float32),
            pltpu.VMEM((G, D), jnp.float32),
            pltpu.VMEM((G, D), jnp.float32),
            pltpu.VMEM((G, D), jnp.float32),
            pltpu.VMEM((G, D), jnp.float32),
            pltpu.SemaphoreType.DMA,
            pltpu.SemaphoreType.DMA,
            pltpu.SemaphoreType.DMA,
            pltpu.SemaphoreType.DMA,
            pltpu.SemaphoreType.DMA,
            pltpu.SemaphoreType.DMA,
            pltpu.SemaphoreType.DMA,
            pltpu.SemaphoreType.DMA,
        ],
    )


def kernel(ctrl_tokens, embed_table, W, b):
    Bc, Tc, _ = ctrl_tokens.shape
    ft01, ft23 = _make_tables(embed_table, W, b.reshape(1, D))
    tf = ctrl_tokens.reshape(-1, 4)
    out = _sc_lookup()(tf[:, 0], tf[:, 1], tf[:, 2], tf[:, 3],
                       ft01.reshape(V * V, D), ft23.reshape(V * V, D))
    return out.reshape(Bc, Tc, D)
